# baseline (speedup 1.0000x reference)
import functools

import jax
import jax.numpy as jnp
from jax import lax
from jax.experimental import pallas as pl
from jax.experimental.pallas import tpu as pltpu
from jax.experimental.pallas import tpu_sc as plsc


def _pre_body(pb_ref, x_ref, o_ref):
    o_ref[...] = lax.dot_general(
        pb_ref[...], x_ref[...],
        dimension_numbers=(((0,), (1,)), ((), ())),
        preferred_element_type=jnp.float32)


def _post_body(sc_ref, n2g_ref, qt_ref, g1t_ref, g2t_ref, bias_ref, lnw_ref,
               lnb_ref, gb_ref, o_ref):
    y = sc_ref[...] + bias_ref[...]
    mu = jnp.mean(y, axis=0, keepdims=True)
    d = y - mu
    var = jnp.mean(d * d, axis=0, keepdims=True)
    yn = d * lax.rsqrt(var + 1e-5)
    yn = yn * lnw_ref[...] + lnb_ref[...]
    zt = lax.dot_general(g2t_ref[...], qt_ref[...],
                         dimension_numbers=(((1,), (0,)), ((), ())),
                         preferred_element_type=jnp.float32)
    ng = qt_ref.shape[1]
    iota = lax.broadcasted_iota(jnp.int32, (ng, sc_ref.shape[1]), 0)
    oh = (n2g_ref[...] == iota).astype(jnp.float32)
    qg = lax.dot_general(zt, oh,
                         dimension_numbers=(((1,), (0,)), ((), ())),
                         preferred_element_type=jnp.float32)
    z1 = lax.dot_general(g1t_ref[...], yn,
                         dimension_numbers=(((1,), (0,)), ((), ())),
                         preferred_element_type=jnp.float32)
    gate = jax.nn.sigmoid(z1 + qg + gb_ref[...])
    o_ref[...] = yn * gate


def _make_sc_kernel(NP, E, CH):
    NCK = E // CH
    mesh = plsc.VectorSubcoreMesh(core_axis_name="c", subcore_axis_name="s")

    @functools.partial(
        pl.kernel,
        out_type=jax.ShapeDtypeStruct((128, NP), jnp.float32),
        mesh=mesh,
        compiler_params=pltpu.CompilerParams(needs_layout_passes=False),
        scratch_types=(
            [pltpu.VMEM((NP,), jnp.float32)] * 4 +
            [pltpu.VMEM((NP,), jnp.float32)] * 4 +
            [
                pltpu.VMEM((NP,), jnp.float32),
                pltpu.VMEM((NP,), jnp.float32),
                pltpu.VMEM((NP,), jnp.float32),
                pltpu.VMEM((2 * CH,), jnp.int32),
                pltpu.VMEM((2 * CH,), jnp.int32),
                pltpu.SemaphoreType.DMA,
                pltpu.SemaphoreType.DMA,
            ]
        ),
    )
    def sc_kernel(pre_hbm, src_hbm, dst_hbm, out_hbm,
                  xp0, xp1, xp2, xp3, ac0, ac1, ac2, ac3,
                  as_v, ad_v, den_v, srcb, dstb, sem_s, sem_d):
        xp = [xp0, xp1, xp2, xp3]
        ac = [ac0, ac1, ac2, ac3]
        c = lax.axis_index("c")
        s = lax.axis_index("s")
        wid = c * 16 + s
        row0 = wid * 4
        h = lax.div(wid, 8)

        for r in range(4):
            pltpu.sync_copy(pre_hbm.at[row0 + r, :], xp[r])
        pltpu.sync_copy(pre_hbm.at[128 + h, :], as_v)
        pltpu.sync_copy(pre_hbm.at[136 + h, :], ad_v)

        @plsc.parallel_loop(0, NP // 16, unroll=4)
        def _zero(i):
            z = jnp.zeros((16,), jnp.float32)
            den_v[pl.ds(i * 16, 16)] = z
            for r in range(4):
                ac[r][pl.ds(i * 16, 16)] = z

        def issue(k, slot):
            pltpu.async_copy(src_hbm.at[pl.ds(k * CH, CH)],
                             srcb.at[pl.ds(slot * CH, CH)], sem_s)
            pltpu.async_copy(dst_hbm.at[pl.ds(k * CH, CH)],
                             dstb.at[pl.ds(slot * CH, CH)], sem_d)

        def wait(k, slot):
            pltpu.make_async_copy(src_hbm.at[pl.ds(k * CH, CH)],
                                  srcb.at[pl.ds(slot * CH, CH)], sem_s).wait()
            pltpu.make_async_copy(dst_hbm.at[pl.ds(k * CH, CH)],
                                  dstb.at[pl.ds(slot * CH, CH)], sem_d).wait()

        issue(0, 0)

        @pl.loop(0, NCK)
        def _chunk(k):
            slot = lax.rem(k, 2)

            @pl.when(k + 1 < NCK)
            def _():
                issue(k + 1, lax.rem(k + 1, 2))

            wait(k, slot)
            soff = slot * CH

            @plsc.parallel_loop(0, CH // 16, unroll=4)
            def _grp(g):
                off = soff + g * 16
                si = srcb[pl.ds(off, 16)]
                di = dstb[pl.ds(off, 16)]
                av = plsc.load_gather(as_v, [si])
                bv = plsc.load_gather(ad_v, [di])
                e = av + bv
                e = jnp.maximum(e, 0.2 * e)
                ex = jnp.exp(e)
                plsc.addupdate_scatter(den_v, [di], ex)
                for r in range(4):
                    xv = plsc.load_gather(xp[r], [si])
                    plsc.addupdate_scatter(ac[r], [di], ex * xv)

        @plsc.parallel_loop(0, NP // 16, unroll=4)
        def _fin(i):
            sl = pl.ds(i * 16, 16)
            inv = 1.0 / (den_v[sl] + 1e-16)
            for r in range(4):
                ac[r][sl] = ac[r][sl] * inv

        for r in range(4):
            pltpu.sync_copy(ac[r], out_hbm.at[row0 + r, :])

    return sc_kernel


def kernel(node_input, edge_index, node2graph, query, W, att_src, att_dst,
           bias, ln_w, ln_b, gate_W, gate_b):
    N, D = node_input.shape
    E = edge_index.shape[1]
    H, DH = att_src.shape
    NG = query.shape[0]
    BN = 2048
    NP = -(-N // BN) * BN
    CH = 2000

    f32 = jnp.float32
    x_pad = jnp.pad(node_input.astype(f32), ((0, NP - N), (0, 0)))
    src = edge_index[0].astype(jnp.int32)
    dst = edge_index[1].astype(jnp.int32)

    eyeH = jnp.eye(H, dtype=f32)
    As = (att_src.astype(f32)[:, :, None] * eyeH[:, None, :]).reshape(D, H)
    Ad = (att_dst.astype(f32)[:, :, None] * eyeH[:, None, :]).reshape(D, H)
    As8 = jnp.pad(W.astype(f32) @ As, ((0, 0), (0, 8 - H)))
    Ad8 = jnp.pad(W.astype(f32) @ Ad, ((0, 0), (0, 8 - H)))
    PB = jnp.concatenate([W.astype(f32), As8, Ad8], axis=1)

    pre = pl.pallas_call(
        _pre_body,
        grid=(NP // BN,),
        in_specs=[
            pl.BlockSpec((D, D + 16), lambda j: (0, 0)),
            pl.BlockSpec((BN, D), lambda j: (j, 0)),
        ],
        out_specs=pl.BlockSpec((D + 16, BN), lambda j: (0, j)),
        out_shape=jax.ShapeDtypeStruct((D + 16, NP), f32),
    )(PB, x_pad)

    outT = _make_sc_kernel(NP, E, CH)(pre, src, dst)

    n2g_p = jnp.pad(node2graph.astype(jnp.int32), (0, NP - N)).reshape(1, NP)
    qT = query.astype(f32).T
    g1t = gate_W[:D].astype(f32).T
    g2t = gate_W[D:].astype(f32).T
    bias2 = bias.astype(f32).reshape(D, 1)
    lnw2 = ln_w.astype(f32).reshape(D, 1)
    lnb2 = ln_b.astype(f32).reshape(D, 1)
    gb2 = gate_b.astype(f32).reshape(D, 1)

    post = pl.pallas_call(
        _post_body,
        grid=(NP // BN,),
        in_specs=[
            pl.BlockSpec((D, BN), lambda j: (0, j)),
            pl.BlockSpec((1, BN), lambda j: (0, j)),
            pl.BlockSpec((D, NG), lambda j: (0, 0)),
            pl.BlockSpec((D, D), lambda j: (0, 0)),
            pl.BlockSpec((D, D), lambda j: (0, 0)),
            pl.BlockSpec((D, 1), lambda j: (0, 0)),
            pl.BlockSpec((D, 1), lambda j: (0, 0)),
            pl.BlockSpec((D, 1), lambda j: (0, 0)),
            pl.BlockSpec((D, 1), lambda j: (0, 0)),
        ],
        out_specs=pl.BlockSpec((D, BN), lambda j: (0, j)),
        out_shape=jax.ShapeDtypeStruct((D, NP), f32),
    )(outT, n2g_p, qT, g1t, g2t, bias2, lnw2, lnb2, gb2)

    return post[:, :N].T


# speedup vs baseline: 1.0176x; 1.0176x over previous
import functools

import jax
import jax.numpy as jnp
from jax import lax
from jax.experimental import pallas as pl
from jax.experimental.pallas import tpu as pltpu
from jax.experimental.pallas import tpu_sc as plsc


def _pre_body(pb_ref, x_ref, o_ref):
    o_ref[...] = lax.dot_general(
        pb_ref[...], x_ref[...],
        dimension_numbers=(((0,), (1,)), ((), ())),
        preferred_element_type=jnp.float32)


def _post_body(sc_ref, n2g_ref, qt_ref, g1t_ref, g2t_ref, bias_ref, lnw_ref,
               lnb_ref, gb_ref, o_ref):
    y = sc_ref[...] + bias_ref[...]
    mu = jnp.mean(y, axis=0, keepdims=True)
    d = y - mu
    var = jnp.mean(d * d, axis=0, keepdims=True)
    yn = d * lax.rsqrt(var + 1e-5)
    yn = yn * lnw_ref[...] + lnb_ref[...]
    zt = lax.dot_general(g2t_ref[...], qt_ref[...],
                         dimension_numbers=(((1,), (0,)), ((), ())),
                         preferred_element_type=jnp.float32)
    ng = qt_ref.shape[1]
    iota = lax.broadcasted_iota(jnp.int32, (ng, sc_ref.shape[1]), 0)
    oh = (n2g_ref[...] == iota).astype(jnp.float32)
    qg = lax.dot_general(zt, oh,
                         dimension_numbers=(((1,), (0,)), ((), ())),
                         preferred_element_type=jnp.float32)
    z1 = lax.dot_general(g1t_ref[...], yn,
                         dimension_numbers=(((1,), (0,)), ((), ())),
                         preferred_element_type=jnp.float32)
    gate = jax.nn.sigmoid(z1 + qg + gb_ref[...])
    o_ref[...] = yn * gate


def _make_sc_kernel(NP, E, CH):
    NCK = E // CH
    mesh = plsc.VectorSubcoreMesh(core_axis_name="c", subcore_axis_name="s")

    @functools.partial(
        pl.kernel,
        out_type=jax.ShapeDtypeStruct((128, NP), jnp.float32),
        mesh=mesh,
        compiler_params=pltpu.CompilerParams(needs_layout_passes=False),
        scratch_types=(
            [pltpu.VMEM((NP,), jnp.float32)] * 4 +
            [pltpu.VMEM((NP,), jnp.float32)] * 4 +
            [
                pltpu.VMEM((NP,), jnp.float32),
                pltpu.VMEM((NP,), jnp.float32),
                pltpu.VMEM((NP,), jnp.float32),
                pltpu.VMEM((2 * CH,), jnp.int32),
                pltpu.VMEM((2 * CH,), jnp.int32),
                pltpu.SemaphoreType.DMA,
                pltpu.SemaphoreType.DMA,
            ]
        ),
    )
    def sc_kernel(pre_hbm, src_hbm, dst_hbm, out_hbm,
                  xp0, xp1, xp2, xp3, ac0, ac1, ac2, ac3,
                  as_v, ad_v, den_v, srcb, dstb, sem_s, sem_d):
        xp = [xp0, xp1, xp2, xp3]
        ac = [ac0, ac1, ac2, ac3]
        c = lax.axis_index("c")
        s = lax.axis_index("s")
        wid = c * 16 + s
        row0 = wid * 4
        h = lax.div(wid, 8)

        for r in range(4):
            pltpu.sync_copy(pre_hbm.at[row0 + r, :], xp[r])
        pltpu.sync_copy(pre_hbm.at[128 + h, :], as_v)
        pltpu.sync_copy(pre_hbm.at[136 + h, :], ad_v)

        @plsc.parallel_loop(0, NP // 16, unroll=4)
        def _zero(i):
            z = jnp.zeros((16,), jnp.float32)
            den_v[pl.ds(i * 16, 16)] = z
            for r in range(4):
                ac[r][pl.ds(i * 16, 16)] = z

        def issue(k, slot):
            pltpu.async_copy(src_hbm.at[pl.ds(k * CH, CH)],
                             srcb.at[pl.ds(slot * CH, CH)], sem_s)
            pltpu.async_copy(dst_hbm.at[pl.ds(k * CH, CH)],
                             dstb.at[pl.ds(slot * CH, CH)], sem_d)

        def wait(k, slot):
            pltpu.make_async_copy(src_hbm.at[pl.ds(k * CH, CH)],
                                  srcb.at[pl.ds(slot * CH, CH)], sem_s).wait()
            pltpu.make_async_copy(dst_hbm.at[pl.ds(k * CH, CH)],
                                  dstb.at[pl.ds(slot * CH, CH)], sem_d).wait()

        issue(0, 0)

        @pl.loop(0, NCK)
        def _chunk(k):
            slot = lax.rem(k, 2)

            @pl.when(k + 1 < NCK)
            def _():
                issue(k + 1, lax.rem(k + 1, 2))

            wait(k, slot)
            soff = slot * CH

            @plsc.parallel_loop(0, CH // 16, unroll=2)
            def _grp(g):
                off = soff + g * 16
                si = srcb[pl.ds(off, 16)]
                di = dstb[pl.ds(off, 16)]
                av = plsc.load_gather(as_v, [si])
                bv = plsc.load_gather(ad_v, [di])
                e = av + bv
                e = jnp.maximum(e, 0.2 * e)
                ex = jnp.exp(e)
                plsc.addupdate_scatter(den_v, [di], ex)
                for r in range(4):
                    xv = plsc.load_gather(xp[r], [si])
                    plsc.addupdate_scatter(ac[r], [di], ex * xv)

        @plsc.parallel_loop(0, NP // 16, unroll=4)
        def _fin(i):
            sl = pl.ds(i * 16, 16)
            inv = 1.0 / (den_v[sl] + 1e-16)
            for r in range(4):
                ac[r][sl] = ac[r][sl] * inv

        for r in range(4):
            pltpu.sync_copy(ac[r], out_hbm.at[row0 + r, :])

    return sc_kernel


def kernel(node_input, edge_index, node2graph, query, W, att_src, att_dst,
           bias, ln_w, ln_b, gate_W, gate_b):
    N, D = node_input.shape
    E = edge_index.shape[1]
    H, DH = att_src.shape
    NG = query.shape[0]
    BN = 2048
    NP = -(-N // BN) * BN
    CH = 2000

    f32 = jnp.float32
    x_pad = jnp.pad(node_input.astype(f32), ((0, NP - N), (0, 0)))
    src = edge_index[0].astype(jnp.int32)
    dst = edge_index[1].astype(jnp.int32)

    eyeH = jnp.eye(H, dtype=f32)
    As = (att_src.astype(f32)[:, :, None] * eyeH[:, None, :]).reshape(D, H)
    Ad = (att_dst.astype(f32)[:, :, None] * eyeH[:, None, :]).reshape(D, H)
    As8 = jnp.pad(W.astype(f32) @ As, ((0, 0), (0, 8 - H)))
    Ad8 = jnp.pad(W.astype(f32) @ Ad, ((0, 0), (0, 8 - H)))
    PB = jnp.concatenate([W.astype(f32), As8, Ad8], axis=1)

    pre = pl.pallas_call(
        _pre_body,
        grid=(NP // BN,),
        in_specs=[
            pl.BlockSpec((D, D + 16), lambda j: (0, 0)),
            pl.BlockSpec((BN, D), lambda j: (j, 0)),
        ],
        out_specs=pl.BlockSpec((D + 16, BN), lambda j: (0, j)),
        out_shape=jax.ShapeDtypeStruct((D + 16, NP), f32),
    )(PB, x_pad)

    outT = _make_sc_kernel(NP, E, CH)(pre, src, dst)

    n2g_p = jnp.pad(node2graph.astype(jnp.int32), (0, NP - N)).reshape(1, NP)
    qT = query.astype(f32).T
    g1t = gate_W[:D].astype(f32).T
    g2t = gate_W[D:].astype(f32).T
    bias2 = bias.astype(f32).reshape(D, 1)
    lnw2 = ln_w.astype(f32).reshape(D, 1)
    lnb2 = ln_b.astype(f32).reshape(D, 1)
    gb2 = gate_b.astype(f32).reshape(D, 1)

    post = pl.pallas_call(
        _post_body,
        grid=(NP // BN,),
        in_specs=[
            pl.BlockSpec((D, BN), lambda j: (0, j)),
            pl.BlockSpec((1, BN), lambda j: (0, j)),
            pl.BlockSpec((D, NG), lambda j: (0, 0)),
            pl.BlockSpec((D, D), lambda j: (0, 0)),
            pl.BlockSpec((D, D), lambda j: (0, 0)),
            pl.BlockSpec((D, 1), lambda j: (0, 0)),
            pl.BlockSpec((D, 1), lambda j: (0, 0)),
            pl.BlockSpec((D, 1), lambda j: (0, 0)),
            pl.BlockSpec((D, 1), lambda j: (0, 0)),
        ],
        out_specs=pl.BlockSpec((D, BN), lambda j: (0, j)),
        out_shape=jax.ShapeDtypeStruct((D, NP), f32),
    )(outT, n2g_p, qT, g1t, g2t, bias2, lnw2, lnb2, gb2)

    return post[:, :N].T


# speedup vs baseline: 1.0521x; 1.0338x over previous
import functools

import jax
import jax.numpy as jnp
from jax import lax
from jax.experimental import pallas as pl
from jax.experimental.pallas import tpu as pltpu
from jax.experimental.pallas import tpu_sc as plsc


def _pre_body(pb_ref, x_ref, o_ref):
    o_ref[...] = lax.dot_general(
        pb_ref[...], x_ref[...],
        dimension_numbers=(((0,), (1,)), ((), ())),
        preferred_element_type=jnp.float32)


def _post_body(sc_ref, n2g_ref, qt_ref, g1t_ref, g2t_ref, bias_ref, lnw_ref,
               lnb_ref, gb_ref, o_ref):
    y = sc_ref[...] + bias_ref[...]
    mu = jnp.mean(y, axis=0, keepdims=True)
    d = y - mu
    var = jnp.mean(d * d, axis=0, keepdims=True)
    yn = d * lax.rsqrt(var + 1e-5)
    yn = yn * lnw_ref[...] + lnb_ref[...]
    zt = lax.dot_general(g2t_ref[...], qt_ref[...],
                         dimension_numbers=(((1,), (0,)), ((), ())),
                         preferred_element_type=jnp.float32)
    ng = qt_ref.shape[1]
    iota = lax.broadcasted_iota(jnp.int32, (ng, sc_ref.shape[1]), 0)
    oh = (n2g_ref[...] == iota).astype(jnp.float32)
    qg = lax.dot_general(zt, oh,
                         dimension_numbers=(((1,), (0,)), ((), ())),
                         preferred_element_type=jnp.float32)
    z1 = lax.dot_general(g1t_ref[...], yn,
                         dimension_numbers=(((1,), (0,)), ((), ())),
                         preferred_element_type=jnp.float32)
    gate = jax.nn.sigmoid(z1 + qg + gb_ref[...])
    o_ref[...] = yn * gate


def _make_sc_kernel(NP, E, CH):
    NCK = E // CH
    mesh = plsc.VectorSubcoreMesh(core_axis_name="c", subcore_axis_name="s")

    @functools.partial(
        pl.kernel,
        out_type=jax.ShapeDtypeStruct((128, NP), jnp.float32),
        mesh=mesh,
        compiler_params=pltpu.CompilerParams(needs_layout_passes=False),
        scratch_types=(
            [pltpu.VMEM((NP,), jnp.float32)] * 4 +
            [pltpu.VMEM((NP,), jnp.float32)] * 4 +
            [
                pltpu.VMEM((NP,), jnp.float32),
                pltpu.VMEM((NP,), jnp.float32),
                pltpu.VMEM((NP,), jnp.float32),
                pltpu.VMEM((2 * CH,), jnp.int32),
                pltpu.SemaphoreType.DMA,
            ]
        ),
    )
    def sc_kernel(pre_hbm, edges_hbm, out_hbm,
                  xp0, xp1, xp2, xp3, ac0, ac1, ac2, ac3,
                  as_v, ad_v, den_v, edb, sem_e):
        xp = [xp0, xp1, xp2, xp3]
        ac = [ac0, ac1, ac2, ac3]
        c = lax.axis_index("c")
        s = lax.axis_index("s")
        wid = c * 16 + s
        row0 = wid * 4
        h = lax.div(wid, 8)

        for r in range(4):
            pltpu.sync_copy(pre_hbm.at[row0 + r, :], xp[r])
        pltpu.sync_copy(pre_hbm.at[128 + h, :], as_v)
        pltpu.sync_copy(pre_hbm.at[136 + h, :], ad_v)

        @plsc.parallel_loop(0, NP // 16, unroll=4)
        def _zero(i):
            z = jnp.zeros((16,), jnp.float32)
            den_v[pl.ds(i * 16, 16)] = z
            for r in range(4):
                ac[r][pl.ds(i * 16, 16)] = z

        def issue(k, slot):
            pltpu.async_copy(edges_hbm.at[pl.ds(k * CH, CH)],
                             edb.at[pl.ds(slot * CH, CH)], sem_e)

        def wait(k, slot):
            pltpu.make_async_copy(edges_hbm.at[pl.ds(k * CH, CH)],
                                  edb.at[pl.ds(slot * CH, CH)], sem_e).wait()

        issue(0, 0)

        @pl.loop(0, NCK)
        def _chunk(k):
            slot = lax.rem(k, 2)

            @pl.when(k + 1 < NCK)
            def _():
                issue(k + 1, lax.rem(k + 1, 2))

            wait(k, slot)
            soff = slot * CH

            @plsc.parallel_loop(0, CH // 16, unroll=2)
            def _grp(g):
                off = soff + g * 16
                w = edb[pl.ds(off, 16)]
                si = lax.bitwise_and(w, 0xFFFF)
                di = lax.shift_right_logical(w, 16)
                av = plsc.load_gather(as_v, [si])
                bv = plsc.load_gather(ad_v, [di])
                e = av + bv
                e = jnp.maximum(e, 0.2 * e)
                ex = jnp.exp(e)
                plsc.addupdate_scatter(den_v, [di], ex)
                for r in range(4):
                    xv = plsc.load_gather(xp[r], [si])
                    plsc.addupdate_scatter(ac[r], [di], ex * xv)

        @plsc.parallel_loop(0, NP // 16, unroll=4)
        def _fin(i):
            sl = pl.ds(i * 16, 16)
            inv = 1.0 / (den_v[sl] + 1e-16)
            for r in range(4):
                ac[r][sl] = ac[r][sl] * inv

        for r in range(4):
            pltpu.sync_copy(ac[r], out_hbm.at[row0 + r, :])

    return sc_kernel


def kernel(node_input, edge_index, node2graph, query, W, att_src, att_dst,
           bias, ln_w, ln_b, gate_W, gate_b):
    N, D = node_input.shape
    E = edge_index.shape[1]
    H, DH = att_src.shape
    NG = query.shape[0]
    BN = 2048
    NP = -(-N // BN) * BN
    CH = 2000

    f32 = jnp.float32
    x_pad = jnp.pad(node_input.astype(f32), ((0, NP - N), (0, 0)))
    src = edge_index[0].astype(jnp.int32)
    dst = edge_index[1].astype(jnp.int32)
    edges_packed = src | (dst << 16)

    eyeH = jnp.eye(H, dtype=f32)
    As = (att_src.astype(f32)[:, :, None] * eyeH[:, None, :]).reshape(D, H)
    Ad = (att_dst.astype(f32)[:, :, None] * eyeH[:, None, :]).reshape(D, H)
    As8 = jnp.pad(W.astype(f32) @ As, ((0, 0), (0, 8 - H)))
    Ad8 = jnp.pad(W.astype(f32) @ Ad, ((0, 0), (0, 8 - H)))
    PB = jnp.concatenate([W.astype(f32), As8, Ad8], axis=1)

    pre = pl.pallas_call(
        _pre_body,
        grid=(NP // BN,),
        in_specs=[
            pl.BlockSpec((D, D + 16), lambda j: (0, 0)),
            pl.BlockSpec((BN, D), lambda j: (j, 0)),
        ],
        out_specs=pl.BlockSpec((D + 16, BN), lambda j: (0, j)),
        out_shape=jax.ShapeDtypeStruct((D + 16, NP), f32),
    )(PB, x_pad)

    outT = _make_sc_kernel(NP, E, CH)(pre, edges_packed)

    n2g_p = jnp.pad(node2graph.astype(jnp.int32), (0, NP - N)).reshape(1, NP)
    qT = query.astype(f32).T
    g1t = gate_W[:D].astype(f32).T
    g2t = gate_W[D:].astype(f32).T
    bias2 = bias.astype(f32).reshape(D, 1)
    lnw2 = ln_w.astype(f32).reshape(D, 1)
    lnb2 = ln_b.astype(f32).reshape(D, 1)
    gb2 = gate_b.astype(f32).reshape(D, 1)

    post = pl.pallas_call(
        _post_body,
        grid=(NP // BN,),
        in_specs=[
            pl.BlockSpec((D, BN), lambda j: (0, j)),
            pl.BlockSpec((1, BN), lambda j: (0, j)),
            pl.BlockSpec((D, NG), lambda j: (0, 0)),
            pl.BlockSpec((D, D), lambda j: (0, 0)),
            pl.BlockSpec((D, D), lambda j: (0, 0)),
            pl.BlockSpec((D, 1), lambda j: (0, 0)),
            pl.BlockSpec((D, 1), lambda j: (0, 0)),
            pl.BlockSpec((D, 1), lambda j: (0, 0)),
            pl.BlockSpec((D, 1), lambda j: (0, 0)),
        ],
        out_specs=pl.BlockSpec((D, BN), lambda j: (0, j)),
        out_shape=jax.ShapeDtypeStruct((D, NP), f32),
    )(outT, n2g_p, qT, g1t, g2t, bias2, lnw2, lnb2, gb2)

    return post[:, :N].T
